# baseline (device time: 46337 ns/iter reference)
import jax
import jax.numpy as jnp
from jax import lax
from jax.experimental import pallas as pl
from jax.experimental.pallas import tpu as pltpu


def kernel(Q, K, V):
    b, s, h, d = Q.shape
    scale = d ** -0.5

    def body(q_ref, k_ref, v_ref, o_ref, kf_ref, vf_ref, send_sems, recv_sems):
        my_x = lax.axis_index("x")
        my_y = lax.axis_index("y")
        my_z = lax.axis_index("z")
        partner = (my_x, 1 - my_y, my_z)

        kf_ref[my_y] = k_ref[...].astype(jnp.bfloat16)
        vf_ref[my_y] = v_ref[...].astype(jnp.bfloat16)

        barrier = pltpu.get_barrier_semaphore()
        pl.semaphore_signal(
            barrier, inc=1, device_id=partner,
            device_id_type=pl.DeviceIdType.MESH,
        )
        pl.semaphore_wait(barrier, 1)

        k_rdma = pltpu.make_async_remote_copy(
            src_ref=kf_ref.at[my_y],
            dst_ref=kf_ref.at[my_y],
            send_sem=send_sems.at[0],
            recv_sem=recv_sems.at[0],
            device_id=partner,
            device_id_type=pl.DeviceIdType.MESH,
        )
        v_rdma = pltpu.make_async_remote_copy(
            src_ref=vf_ref.at[my_y],
            dst_ref=vf_ref.at[my_y],
            send_sem=send_sems.at[1],
            recv_sem=recv_sems.at[1],
            device_id=partner,
            device_id_type=pl.DeviceIdType.MESH,
        )
        k_rdma.start()
        v_rdma.start()
        k_rdma.wait()
        v_rdma.wait()

        for bi in range(b):
            for hi in range(h):
                q = q_ref[bi, :, hi, :].astype(jnp.bfloat16)
                kk = jnp.concatenate(
                    [kf_ref[0, bi, :, hi, :], kf_ref[1, bi, :, hi, :]], axis=0
                )
                vv = jnp.concatenate(
                    [vf_ref[0, bi, :, hi, :], vf_ref[1, bi, :, hi, :]], axis=0
                )
                s_mat = lax.dot_general(
                    q, kk, (((1,), (1,)), ((), ())),
                    preferred_element_type=jnp.float32,
                ) * scale
                m = jnp.max(s_mat, axis=1, keepdims=True)
                p = jnp.exp(s_mat - m)
                l = jnp.sum(p, axis=1, keepdims=True)
                p = (p / l).astype(jnp.bfloat16)
                o = lax.dot_general(
                    p, vv, (((1,), (0,)), ((), ())),
                    preferred_element_type=jnp.float32,
                )
                o_ref[bi, :, hi, :] = o

    return pl.pallas_call(
        body,
        out_shape=jax.ShapeDtypeStruct((b, s, h, d), jnp.float32),
        in_specs=[
            pl.BlockSpec(memory_space=pltpu.VMEM),
            pl.BlockSpec(memory_space=pltpu.VMEM),
            pl.BlockSpec(memory_space=pltpu.VMEM),
        ],
        out_specs=pl.BlockSpec(memory_space=pltpu.VMEM),
        scratch_shapes=[
            pltpu.VMEM((2, b, s, h, d), jnp.bfloat16),
            pltpu.VMEM((2, b, s, h, d), jnp.bfloat16),
            pltpu.SemaphoreType.DMA((2,)),
            pltpu.SemaphoreType.DMA((2,)),
        ],
        compiler_params=pltpu.CompilerParams(collective_id=0),
    )(Q, K, V)


# device time: 20678 ns/iter; 2.2409x vs baseline; 2.2409x over previous
import jax
import jax.numpy as jnp
from jax import lax
from jax.experimental import pallas as pl
from jax.experimental.pallas import tpu as pltpu

N_CHUNK = 8


def kernel(Q, K, V):
    b, s, h, d = Q.shape
    hd = h * d
    scale = d ** -0.5
    n_per_b = N_CHUNK // b
    sc = s // n_per_b

    Q3 = Q.reshape(b, s, hd)
    K3 = K.reshape(b, s, hd)
    V3 = V.reshape(b, s, hd)

    def body(q_hbm, k_hbm, v_hbm, o_hbm, q_ref, k_ref, v_ref, o_vmem,
             qb_ref, kf_ref, vf_ref, o_scr, l_scr,
             ysend, yin, zsend, zin, insem, outsem):
        my_x = lax.axis_index("x")
        my_y = lax.axis_index("y")
        my_z = lax.axis_index("z")
        partner = (my_x, 1 - my_y, my_z)
        zbuddy = (my_x, my_y, 1 - my_z)

        cp_k = pltpu.make_async_copy(k_hbm, k_ref, insem.at[1])
        cp_v = pltpu.make_async_copy(v_hbm, v_ref, insem.at[2])
        cp_q = pltpu.make_async_copy(q_hbm, q_ref, insem.at[0])
        cp_k.start()
        cp_v.start()
        cp_q.start()

        barrier = pltpu.get_barrier_semaphore()
        for nbr in (partner, zbuddy):
            pl.semaphore_signal(
                barrier, inc=1, device_id=nbr,
                device_id_type=pl.DeviceIdType.MESH,
            )

        cp_k.wait()
        cp_v.wait()
        kf_ref[0] = k_ref[...].astype(jnp.bfloat16)
        vf_ref[0] = v_ref[...].astype(jnp.bfloat16)

        pl.semaphore_wait(barrier, 2)

        def chunk_at(ref, slot, ci):
            bi, q = divmod(ci, n_per_b)
            return ref.at[slot, bi, pl.ds(q * sc, sc)]

        def make_y(t_ref, ci):
            return pltpu.make_async_remote_copy(
                src_ref=chunk_at(t_ref, 0, ci),
                dst_ref=chunk_at(t_ref, 1, ci),
                send_sem=ysend.at[ci],
                recv_sem=yin.at[ci],
                device_id=partner,
                device_id_type=pl.DeviceIdType.MESH,
            )

        def make_z(t_ref, ci):
            return pltpu.make_async_remote_copy(
                src_ref=chunk_at(t_ref, 1, ci),
                dst_ref=chunk_at(t_ref, 1, ci),
                send_sem=zsend.at[ci],
                recv_sem=zin.at[ci],
                device_id=zbuddy,
                device_id_type=pl.DeviceIdType.MESH,
            )

        @pl.when(my_z == 0)
        def _():
            for ci in range(N_CHUNK):
                make_y(kf_ref, ci).start()

        @pl.when(my_z == 1)
        def _():
            for ci in range(N_CHUNK):
                make_y(vf_ref, ci).start()

        def service_chunk(ci):
            make_y(kf_ref, ci).wait_recv()

            @pl.when(my_z == 0)
            def _():
                make_z(kf_ref, ci).start()

            @pl.when(my_z == 1)
            def _():
                make_z(vf_ref, ci).start()

        cp_q.wait()
        qb_ref[...] = (q_ref[...] * scale).astype(jnp.bfloat16)
        ones = jnp.ones((s, 1), jnp.bfloat16)

        def attn_block(bi, hi, slot):
            cols = pl.ds(hi * d, d)
            q = qb_ref[bi, :, cols]
            kk = kf_ref[slot, bi, :, cols]
            vv = vf_ref[slot, bi, :, cols]
            s_mat = lax.dot_general(
                q, kk, (((1,), (1,)), ((), ())),
                preferred_element_type=jnp.float32,
            )
            p = jnp.exp(s_mat).astype(jnp.bfloat16)
            l = lax.dot_general(
                p, ones, (((1,), (0,)), ((), ())),
                preferred_element_type=jnp.float32,
            )
            o = lax.dot_general(
                p, vv, (((1,), (0,)), ((), ())),
                preferred_element_type=jnp.float32,
            )
            return l, o

        pairs = [(bi, hi) for bi in range(b) for hi in range(h)]
        for idx, (bi, hi) in enumerate(pairs):
            l1, o1 = attn_block(bi, hi, 0)
            cols = pl.ds(hi * d, d)
            o_scr[bi, :, cols] = o1
            l_scr[bi, :, pl.ds(hi, 1)] = l1
            if idx % (len(pairs) // N_CHUNK) == 1:
                service_chunk(idx // (len(pairs) // N_CHUNK))

        for bi in range(b):
            for ci in range(bi * n_per_b, (bi + 1) * n_per_b):
                make_z(kf_ref, ci).wait_recv()
            for hi in range(h):
                l2, o2 = attn_block(bi, hi, 1)
                cols = pl.ds(hi * d, d)
                l1 = l_scr[bi, :, pl.ds(hi, 1)]
                o1 = o_scr[bi, :, cols]
                o_vmem[bi, :, cols] = (o1 + o2) / (l1 + l2)
            pltpu.make_async_copy(
                o_vmem.at[bi], o_hbm.at[bi], outsem.at[bi]
            ).start()

        for bi in range(b):
            pltpu.make_async_copy(
                o_vmem.at[bi], o_hbm.at[bi], outsem.at[bi]
            ).wait()

        for ci in range(N_CHUNK):
            make_y(kf_ref, ci).wait_send()
            make_z(kf_ref, ci).wait_send()

    out3 = pl.pallas_call(
        body,
        out_shape=jax.ShapeDtypeStruct((b, s, hd), jnp.float32),
        in_specs=[
            pl.BlockSpec(memory_space=pl.ANY),
            pl.BlockSpec(memory_space=pl.ANY),
            pl.BlockSpec(memory_space=pl.ANY),
        ],
        out_specs=pl.BlockSpec(memory_space=pl.ANY),
        scratch_shapes=[
            pltpu.VMEM((b, s, hd), jnp.float32),
            pltpu.VMEM((b, s, hd), jnp.float32),
            pltpu.VMEM((b, s, hd), jnp.float32),
            pltpu.VMEM((b, s, hd), jnp.float32),
            pltpu.VMEM((b, s, hd), jnp.bfloat16),
            pltpu.VMEM((2, b, s, hd), jnp.bfloat16),
            pltpu.VMEM((2, b, s, hd), jnp.bfloat16),
            pltpu.VMEM((b, s, hd), jnp.float32),
            pltpu.VMEM((b, s, h), jnp.float32),
            pltpu.SemaphoreType.DMA((N_CHUNK,)),
            pltpu.SemaphoreType.DMA((N_CHUNK,)),
            pltpu.SemaphoreType.DMA((N_CHUNK,)),
            pltpu.SemaphoreType.DMA((N_CHUNK,)),
            pltpu.SemaphoreType.DMA((3,)),
            pltpu.SemaphoreType.DMA((b,)),
        ],
        compiler_params=pltpu.CompilerParams(collective_id=0),
    )(Q3, K3, V3)
    return out3.reshape(b, s, h, d)
